# baseline (speedup 1.0000x reference)
import functools

import jax
import jax.numpy as jnp
from jax import lax
from jax.experimental import pallas as pl
from jax.experimental.pallas import tpu as pltpu
from jax.experimental.pallas import tpu_sc as plsc

N = 10000
E = 320000
G = 16

NC = 2
NS = 16
NW = NC * NS
EPW = E // NW
B = 40
CH = EPW // B
CH4 = (CH // 4) * 4
NPAD = 10240
NPS = NPAD // NS
ZR = NPS // 5


def _a1_body(nfp_ref, wsp_ref, wdp_ref, batch_ref, gpb_ref, ps_ref, pd_ref):
    nfp = nfp_ref[...]
    oh = (batch_ref[...] == lax.broadcasted_iota(jnp.int32, (1, G), 1)
          ).astype(jnp.float32)
    ps_ref[...] = (jnp.dot(nfp, wsp_ref[...], preferred_element_type=jnp.float32)
                   + jnp.dot(oh, gpb_ref[...], preferred_element_type=jnp.float32))
    pd_ref[...] = jnp.dot(nfp, wdp_ref[...], preferred_element_type=jnp.float32)


EB = 8000


def _a2_body(ea_ref, src_ref, starts_ref, w2_ref, base_ref, bsrc_ref, ecnt_ref):
    src = src_ref[0, 0, :]
    bsrc = jnp.zeros((EB,), jnp.int32)
    for g in range(1, G):
        bsrc = bsrc + (src >= starts_ref[0, g]).astype(jnp.int32)
    bsrc_ref[0, 0, :] = bsrc
    oh = (bsrc[:, None] == lax.broadcasted_iota(jnp.int32, (EB, G), 1)
          ).astype(jnp.float32)
    base_ref[...] = jnp.dot(ea_ref[...], w2_ref[...],
                            preferred_element_type=jnp.float32)

    @pl.when(pl.program_id(0) == 0)
    def _():
        ecnt_ref[...] = jnp.zeros_like(ecnt_ref)

    ecnt_ref[0:1, 0:G] += jnp.sum(oh, axis=0, keepdims=True)


def _sc_body(ps_hbm, pd_hbm, base_hbm, idx_hbm,
             ef_out, agg_out, deg_out, gagg_out,
             ib0, ib1, ib2, ib3,
             rs0, rs1, rd0, rd1, rb0, rb1,
             ones_v, degbuf,
             agg_sh, deg_sh, gagg_sh,
             si0, si1, si2, si3,
             sgs0, sgs1, sgd0, sgd1, sgb0, sgb1,
             sef0, sef1, sag0, sag1, sdg0, sdg1, sgg0, sgg1):
    c = lax.axis_index("c")
    s = lax.axis_index("s")
    wid = s * NC + c

    ib = [ib0, ib1, ib2, ib3]
    rs = [rs0, rs1]
    rd = [rd0, rd1]
    rb = [rb0, rb1]
    si = [si0, si1, si2, si3]
    sgs = [sgs0, sgs1]
    sgd = [sgd0, sgd1]
    sgb = [sgb0, sgb1]
    sef = [sef0, sef1]
    sag = [sag0, sag1]
    sdg = [sdg0, sdg1]
    sgg = [sgg0, sgg1]

    z16 = jnp.zeros((16,), jnp.float32)

    def _zero_bufs(r, _):
        for cg in range(8):
            rs0[r, pl.ds(cg * 16, 16)] = z16
        degbuf[r, :] = z16
        ones_v[r, :] = jnp.ones((16,), jnp.float32)
        return 0
    lax.fori_loop(0, B, _zero_bufs, 0)

    def _issue_idx(i, islot):
        pltpu.async_copy(idx_hbm.at[wid * CH + i], ib[islot], si[islot])

    def _drain_idx(islot):
        pltpu.make_async_copy(idx_hbm.at[0], ib[islot], si[islot]).wait()

    def _issue_gathers(j, islot, r):
        e0 = wid * EPW + j * B
        pltpu.async_copy(ps_hbm.at[ib[islot].at[0]], rs[r], sgs[r])
        pltpu.async_copy(pd_hbm.at[ib[islot].at[1]], rd[r], sgd[r])
        pltpu.async_copy(base_hbm.at[pl.ds(e0, B)], rb[r], sgb[r])

    def _drain_gathers(r):
        pltpu.make_async_copy(ps_hbm.at[pl.ds(0, B)], rs[r], sgs[r]).wait()
        pltpu.make_async_copy(pd_hbm.at[pl.ds(0, B)], rd[r], sgd[r]).wait()
        pltpu.make_async_copy(base_hbm.at[pl.ds(0, B)], rb[r], sgb[r]).wait()

    def _compute(r):
        def _add_row(row, _):
            for cg in range(8):
                sl = pl.ds(cg * 16, 16)
                rb[r][row, sl] = rb[r][row, sl] + rs[r][row, sl] + rd[r][row, sl]
            return 0
        lax.fori_loop(0, B, _add_row, 0)

    def _issue_stores(i, p, r):
        e0 = wid * EPW + i * B
        pltpu.async_copy(rb[r], ef_out.at[pl.ds(e0, B)], sef[r])
        pltpu.async_copy(rb[r], agg_sh.at[ib[p].at[1]], sag[r], add=True)
        pltpu.async_copy(ones_v, deg_sh.at[ib[p].at[1]], sdg[r], add=True)
        pltpu.async_copy(rb[r], gagg_sh.at[ib[p].at[2]], sgg[r], add=True)

    def _drain_stores(r):
        pltpu.make_async_copy(rb[r], ef_out.at[pl.ds(0, B)], sef[r]).wait()
        pltpu.make_async_copy(rb[r], agg_sh.at[pl.ds(0, B)], sag[r]).wait()
        pltpu.make_async_copy(ones_v, deg_sh.at[pl.ds(0, B)], sdg[r]).wait()
        pltpu.make_async_copy(rb[r], agg_sh.at[pl.ds(0, B)], sgg[r]).wait()

    def _half(i, p, g_prev, g_pf, has_next):
        r = p % 2
        if g_prev is True:
            _drain_stores(1 - r)
        else:
            pl.when(g_prev)(lambda: _drain_stores(1 - r))
        if has_next:
            _drain_idx((p + 1) % 4)
            _issue_gathers(i + 1, (p + 1) % 4, 1 - r)
        if g_pf is True:
            _issue_idx(i + 3, (p + 3) % 4)
        elif g_pf is not False:
            pl.when(g_pf)(lambda: _issue_idx(i + 3, (p + 3) % 4))
        _drain_gathers(r)
        _compute(r)
        _issue_stores(i, p, r)

    _issue_idx(0, 0)
    _issue_idx(1, 1)
    _issue_idx(2, 2)
    for k in range(NPS // B):
        pltpu.sync_copy(rs0, agg_sh.at[pl.ds(s * NPS + k * B, B)])
        pltpu.sync_copy(degbuf, deg_sh.at[pl.ds(s * NPS + k * B, B)])

    @pl.when(s == 0)
    def _():
        pltpu.sync_copy(rs0.at[pl.ds(0, G)], gagg_sh)

    _drain_idx(0)
    _issue_gathers(0, 0, 0)

    plsc.subcore_barrier()

    def _quad(k, _):
        i0 = 4 * k
        _half(i0, 0, i0 > 0, True, True)
        _half(i0 + 1, 1, True, True, True)
        _half(i0 + 2, 2, True, True, True)
        _half(i0 + 3, 3, True, i0 + 6 < CH, True)
        return 0
    lax.fori_loop(0, CH4 // 4, _quad, 0)

    for j in range(CH4, CH):
        _half(j, j % 4, True, False, j + 1 < CH)
    _drain_stores((CH - 1) % 2)

    plsc.subcore_barrier()

    for k in range(NPS // B):
        r0 = s * NPS + k * B
        pltpu.sync_copy(agg_sh.at[pl.ds(r0, B)], rs0)
        pltpu.sync_copy(rs0, agg_out.at[c, pl.ds(r0, B)])
        pltpu.sync_copy(deg_sh.at[pl.ds(r0, B)], degbuf)
        pltpu.sync_copy(degbuf, deg_out.at[c, pl.ds(r0, B)])

    @pl.when(s == 0)
    def _():
        pltpu.sync_copy(gagg_sh, rs0.at[pl.ds(0, G)])
        pltpu.sync_copy(rs0.at[pl.ds(0, G)], gagg_out.at[c])


NB = 2000


def _d_body(agg_ref, deg_ref, x_ref, pos_ref, batch_ref, gf_ref,
            wn1_ref, wnx_ref, wnp_ref, wng_ref, bn_ref,
            gagg_ref, ecnt_ref, ncnt_ref,
            wg1_ref, wg2_ref, wg3_ref, bg_ref,
            unf_ref, nsum_ref, ugf_ref):
    x = x_ref[...]
    pos = pos_ref[...]
    aggsum = agg_ref[0] + agg_ref[1]
    deg = deg_ref[0, :, 0:1] + deg_ref[1, :, 0:1]
    agg_ef = aggsum / jnp.maximum(deg, 1.0)
    bt = batch_ref[0, 0, :]
    oh = (bt[:, None] == lax.broadcasted_iota(jnp.int32, (NB, G), 1)
          ).astype(jnp.float32)
    rep = jnp.dot(oh, gf_ref[...], preferred_element_type=jnp.float32)
    acc = (jnp.dot(agg_ef, wn1_ref[...], preferred_element_type=jnp.float32)
           + jnp.dot(x, wnx_ref[...], preferred_element_type=jnp.float32)
           + jnp.dot(pos, wnp_ref[...], preferred_element_type=jnp.float32)
           + jnp.dot(rep, wng_ref[...], preferred_element_type=jnp.float32)
           + bn_ref[...])
    unf = acc + jnp.concatenate([x, pos], axis=1)
    unf_ref[...] = unf

    @pl.when(pl.program_id(0) == 0)
    def _():
        nsum_ref[...] = jnp.zeros_like(nsum_ref)

    nsum_ref[...] += lax.dot_general(
        oh, unf, (((0,), (0,)), ((), ())),
        preferred_element_type=jnp.float32)

    @pl.when(pl.program_id(0) == pl.num_programs(0) - 1)
    def _():
        agg_nf = nsum_ref[...] / jnp.maximum(ncnt_ref[...], 1.0)
        gaggsum = gagg_ref[0] + gagg_ref[1]
        agg_efg = gaggsum / jnp.maximum(ecnt_ref[...], 1.0)
        gf = gf_ref[...]
        ugf_ref[...] = (
            jnp.dot(agg_nf, wg1_ref[...], preferred_element_type=jnp.float32)
            + jnp.dot(agg_efg, wg2_ref[...], preferred_element_type=jnp.float32)
            + jnp.dot(gf, wg3_ref[...], preferred_element_type=jnp.float32)
            + bg_ref[...] + gf)


def kernel(x, pos, edge_attr, global_feats, W_e, b_e, W_n, b_n, W_g, b_g,
           edge_index, batch):
    f32 = jnp.float32
    src = edge_index[0].astype(jnp.int32)
    dst = edge_index[1].astype(jnp.int32)
    batch = batch.astype(jnp.int32)

    nf_pad = jnp.concatenate(
        [x, pos, jnp.zeros((N, 5), f32)], axis=1)
    Wsp = jnp.concatenate([W_e[0:131], jnp.zeros((5, 128), f32)], axis=0)
    Wdp = jnp.concatenate([W_e[131:262], jnp.zeros((5, 128), f32)], axis=0)
    Gpb = global_feats @ W_e[278:310] + b_e[None, :]
    W2 = W_e[262:278]
    starts = jnp.searchsorted(batch, jnp.arange(G, dtype=jnp.int32)
                              ).astype(jnp.int32)[None, :]
    ncnt = jnp.diff(jnp.concatenate(
        [starts[0], jnp.array([N], jnp.int32)])).astype(f32)[:, None]

    ps, pd = pl.pallas_call(
        _a1_body,
        out_shape=(jax.ShapeDtypeStruct((N, 128), f32),
                   jax.ShapeDtypeStruct((N, 128), f32)),
    )(nf_pad, Wsp, Wdp, batch.reshape(N, 1), Gpb)

    src3 = src.reshape(E // EB, 1, EB)
    grid_a2 = (E // EB,)
    base, bsrc3, ecnt = pl.pallas_call(
        _a2_body,
        grid=grid_a2,
        in_specs=[
            pl.BlockSpec((EB, 16), lambda i: (i, 0)),
            pl.BlockSpec((1, 1, EB), lambda i: (i, 0, 0)),
            pl.BlockSpec((1, G), lambda i: (0, 0)),
            pl.BlockSpec((16, 128), lambda i: (0, 0)),
        ],
        out_specs=[
            pl.BlockSpec((EB, 128), lambda i: (i, 0)),
            pl.BlockSpec((1, 1, EB), lambda i: (i, 0, 0)),
            pl.BlockSpec((8, 128), lambda i: (0, 0)),
        ],
        out_shape=(jax.ShapeDtypeStruct((E, 128), f32),
                   jax.ShapeDtypeStruct((E // EB, 1, EB), jnp.int32),
                   jax.ShapeDtypeStruct((8, 128), f32)),
    )(edge_attr, src3, starts, W2)
    bsrc = bsrc3.reshape(E)
    ecnt_col = ecnt[0, 0:G][:, None]

    idx_pack = jnp.stack([src.reshape(E // B, B), dst.reshape(E // B, B),
                          bsrc.reshape(E // B, B)], axis=1)
    mesh = plsc.VectorSubcoreMesh(core_axis_name="c", subcore_axis_name="s",
                                  num_cores=NC, num_subcores=NS)
    sc_edge = functools.partial(
        pl.kernel, _sc_body,
        compiler_params=pltpu.CompilerParams(use_tc_tiling_on_sc=False),
        out_type=(jax.ShapeDtypeStruct((E, 128), f32),
                  jax.ShapeDtypeStruct((NC, NPAD, 128), f32),
                  jax.ShapeDtypeStruct((NC, NPAD, 16), f32),
                  jax.ShapeDtypeStruct((NC, G, 128), f32)),
        mesh=mesh,
        scratch_types=(
            [pltpu.VMEM((3, B), jnp.int32) for _ in range(4)]
            + [pltpu.VMEM((B, 128), f32) for _ in range(6)]
            + [pltpu.VMEM((B, 16), f32), pltpu.VMEM((B, 16), f32)]
            + [pltpu.VMEM_SHARED((NPAD, 128), f32),
               pltpu.VMEM_SHARED((NPAD, 16), f32),
               pltpu.VMEM_SHARED((G, 128), f32)]
            + [pltpu.SemaphoreType.DMA for _ in range(18)]
        ),
    )()
    updated_ef, agg2, deg2, gagg2 = sc_edge(ps, pd, base, idx_pack)

    batch3 = batch.reshape(N // NB, 1, NB)
    Wn1 = W_n[0:128]
    Wnx = W_n[128:256]
    Wnp = W_n[256:259]
    Wng = W_n[259:291]
    Wg1 = W_g[0:131]
    Wg2 = W_g[131:259]
    Wg3 = W_g[259:291]
    grid_d = (N // NB,)
    rep_spec = lambda shape: pl.BlockSpec(shape, lambda i: tuple(0 for _ in shape))
    updated_nf, _nsum, updated_gf = pl.pallas_call(
        _d_body,
        grid=grid_d,
        in_specs=[
            pl.BlockSpec((NC, NB, 128), lambda i: (0, i, 0)),
            pl.BlockSpec((NC, NB, 16), lambda i: (0, i, 0)),
            pl.BlockSpec((NB, 128), lambda i: (i, 0)),
            pl.BlockSpec((NB, 3), lambda i: (i, 0)),
            pl.BlockSpec((1, 1, NB), lambda i: (i, 0, 0)),
            rep_spec((G, 32)),
            rep_spec((128, 131)),
            rep_spec((128, 131)),
            rep_spec((3, 131)),
            rep_spec((32, 131)),
            rep_spec((1, 131)),
            rep_spec((NC, G, 128)),
            rep_spec((G, 1)),
            rep_spec((G, 1)),
            rep_spec((131, 32)),
            rep_spec((128, 32)),
            rep_spec((32, 32)),
            rep_spec((1, 32)),
        ],
        out_specs=[
            pl.BlockSpec((NB, 131), lambda i: (i, 0)),
            rep_spec((G, 131)),
            rep_spec((G, 32)),
        ],
        out_shape=(jax.ShapeDtypeStruct((N, 131), f32),
                   jax.ShapeDtypeStruct((G, 131), f32),
                   jax.ShapeDtypeStruct((G, 32), f32)),
    )(agg2, deg2, x, pos, batch3, global_feats,
      Wn1, Wnx, Wnp, Wng, b_n[None, :],
      gagg2, ecnt_col, ncnt,
      Wg1, Wg2, Wg3, b_g[None, :])

    return (updated_nf, updated_ef, updated_gf)


# speedup vs baseline: 1.0194x; 1.0194x over previous
import functools

import jax
import jax.numpy as jnp
from jax import lax
from jax.experimental import pallas as pl
from jax.experimental.pallas import tpu as pltpu
from jax.experimental.pallas import tpu_sc as plsc

N = 10000
E = 320000
G = 16

NC = 2
NS = 16
NW = NC * NS
EPW = E // NW
B = 40
CH = EPW // B
CH4 = (CH // 4) * 4
NPAD = 10240
NPS = NPAD // NS
ZR = NPS // 5


def _a1_body(nfp_ref, wsp_ref, wdp_ref, ps_ref, pd_ref):
    nfp = nfp_ref[...]
    ps_ref[...] = jnp.dot(nfp, wsp_ref[...], preferred_element_type=jnp.float32)
    pd_ref[...] = jnp.dot(nfp, wdp_ref[...], preferred_element_type=jnp.float32)


EB = 8000


def _a2_body(ea_ref, src_ref, starts_ref, w2_ref, base_ref, bsrc_ref, ecnt_ref):
    src = src_ref[0, 0, :]
    bsrc = jnp.zeros((EB,), jnp.int32)
    for g in range(1, G):
        bsrc = bsrc + (src >= starts_ref[0, g]).astype(jnp.int32)
    bsrc_ref[0, 0, :] = bsrc
    oh = (bsrc[:, None] == lax.broadcasted_iota(jnp.int32, (EB, G), 1)
          ).astype(jnp.float32)
    base_ref[...] = (
        jnp.dot(ea_ref[...], w2_ref[0:16], preferred_element_type=jnp.float32)
        + jnp.dot(oh, w2_ref[16:32], preferred_element_type=jnp.float32))

    @pl.when(pl.program_id(0) == 0)
    def _():
        ecnt_ref[...] = jnp.zeros_like(ecnt_ref)

    ecnt_ref[0:1, 0:G] += jnp.sum(oh, axis=0, keepdims=True)


def _sc_body(ps_hbm, pd_hbm, base_hbm, idx_hbm,
             ef_out, agg_out, deg_out, gagg_out,
             ib0, ib1, ib2, ib3,
             rs0, rs1, rd0, rd1, rb0, rb1,
             ones_v, degbuf,
             agg_sh, deg_sh, gagg_sh,
             si0, si1, si2, si3,
             sgs0, sgs1, sgd0, sgd1, sgb0, sgb1,
             sef0, sef1, sag0, sag1, sdg0, sdg1, sgg0, sgg1, sz0, sz1):
    c = lax.axis_index("c")
    s = lax.axis_index("s")
    wid = s * NC + c

    ib = [ib0, ib1, ib2, ib3]
    rs = [rs0, rs1]
    rd = [rd0, rd1]
    rb = [rb0, rb1]
    si = [si0, si1, si2, si3]
    sgs = [sgs0, sgs1]
    sgd = [sgd0, sgd1]
    sgb = [sgb0, sgb1]
    sef = [sef0, sef1]
    sag = [sag0, sag1]
    sdg = [sdg0, sdg1]
    sgg = [sgg0, sgg1]

    z16 = jnp.zeros((16,), jnp.float32)

    def _zero_bufs(r, _):
        for cg in range(8):
            rs1[r, pl.ds(cg * 16, 16)] = z16
        degbuf[r, :] = z16
        ones_v[r, :] = jnp.ones((16,), jnp.float32)
        return 0
    lax.fori_loop(0, B, _zero_bufs, 0)

    def _issue_idx(i, islot):
        pltpu.async_copy(idx_hbm.at[wid * CH + i], ib[islot], si[islot])

    def _drain_idx(islot):
        pltpu.make_async_copy(idx_hbm.at[0], ib[islot], si[islot]).wait()

    def _issue_gathers(j, islot, r):
        e0 = wid * EPW + j * B
        pltpu.async_copy(ps_hbm.at[ib[islot].at[0]], rs[r], sgs[r])
        pltpu.async_copy(pd_hbm.at[ib[islot].at[1]], rd[r], sgd[r])
        pltpu.async_copy(base_hbm.at[pl.ds(e0, B)], rb[r], sgb[r])

    def _drain_gathers(r):
        pltpu.make_async_copy(ps_hbm.at[pl.ds(0, B)], rs[r], sgs[r]).wait()
        pltpu.make_async_copy(pd_hbm.at[pl.ds(0, B)], rd[r], sgd[r]).wait()
        pltpu.make_async_copy(base_hbm.at[pl.ds(0, B)], rb[r], sgb[r]).wait()

    def _compute(r):
        def _add_row(row, _):
            for cg in range(8):
                sl = pl.ds(cg * 16, 16)
                rb[r][row, sl] = rb[r][row, sl] + rs[r][row, sl] + rd[r][row, sl]
            return 0
        lax.fori_loop(0, B, _add_row, 0)

    def _issue_stores(i, p, r):
        e0 = wid * EPW + i * B
        pltpu.async_copy(rb[r], ef_out.at[pl.ds(e0, B)], sef[r])
        pltpu.async_copy(rb[r], agg_sh.at[ib[p].at[1]], sag[r], add=True)
        pltpu.async_copy(ones_v, deg_sh.at[ib[p].at[1]], sdg[r], add=True)
        pltpu.async_copy(rb[r], gagg_sh.at[ib[p].at[2]], sgg[r], add=True)

    def _drain_stores(r):
        pltpu.make_async_copy(rb[r], ef_out.at[pl.ds(0, B)], sef[r]).wait()
        pltpu.make_async_copy(rb[r], agg_sh.at[pl.ds(0, B)], sag[r]).wait()
        pltpu.make_async_copy(ones_v, deg_sh.at[pl.ds(0, B)], sdg[r]).wait()
        pltpu.make_async_copy(rb[r], agg_sh.at[pl.ds(0, B)], sgg[r]).wait()

    def _half(i, p, g_prev, g_pf, has_next):
        r = p % 2
        if g_prev is True:
            _drain_stores(1 - r)
        else:
            pl.when(g_prev)(lambda: _drain_stores(1 - r))
        if has_next:
            _drain_idx((p + 1) % 4)
            _issue_gathers(i + 1, (p + 1) % 4, 1 - r)
        if g_pf is True:
            _issue_idx(i + 3, (p + 3) % 4)
        elif g_pf is not False:
            pl.when(g_pf)(lambda: _issue_idx(i + 3, (p + 3) % 4))
        _drain_gathers(r)
        _compute(r)
        _issue_stores(i, p, r)

    _issue_idx(0, 0)
    _issue_idx(1, 1)
    _issue_idx(2, 2)
    _drain_idx(0)
    _issue_gathers(0, 0, 0)

    for k in range(NPS // B):
        pltpu.async_copy(rs1, agg_sh.at[pl.ds(s * NPS + k * B, B)], sz0)
        pltpu.async_copy(degbuf, deg_sh.at[pl.ds(s * NPS + k * B, B)], sz1)

    @pl.when(s == 0)
    def _():
        pltpu.sync_copy(rs1.at[pl.ds(0, G)], gagg_sh)

    for k in range(NPS // B):
        pltpu.make_async_copy(rs1, agg_sh.at[pl.ds(0, B)], sz0).wait()
        pltpu.make_async_copy(degbuf, deg_sh.at[pl.ds(0, B)], sz1).wait()

    plsc.subcore_barrier()

    def _quad(k, _):
        i0 = 4 * k
        _half(i0, 0, i0 > 0, True, True)
        _half(i0 + 1, 1, True, True, True)
        _half(i0 + 2, 2, True, True, True)
        _half(i0 + 3, 3, True, i0 + 6 < CH, True)
        return 0
    lax.fori_loop(0, CH4 // 4, _quad, 0)

    for j in range(CH4, CH):
        _half(j, j % 4, True, False, j + 1 < CH)
    _drain_stores((CH - 1) % 2)

    plsc.subcore_barrier()

    rstage = [rs0, rs1]
    dstage = [degbuf, ones_v]
    for k in range(NPS // B):
        r0 = s * NPS + k * B
        rbuf = rstage[k % 2]
        dbuf = dstage[k % 2]
        if k >= 2:
            pltpu.make_async_copy(rbuf, agg_out.at[0, pl.ds(0, B)],
                                  sz0).wait()
            pltpu.make_async_copy(dbuf, deg_out.at[0, pl.ds(0, B)],
                                  sz1).wait()
        pltpu.sync_copy(agg_sh.at[pl.ds(r0, B)], rbuf)
        pltpu.async_copy(rbuf, agg_out.at[c, pl.ds(r0, B)], sz0)
        pltpu.sync_copy(deg_sh.at[pl.ds(r0, B)], dbuf)
        pltpu.async_copy(dbuf, deg_out.at[c, pl.ds(r0, B)], sz1)
    for k in range(2):
        pltpu.make_async_copy(rs0, agg_out.at[0, pl.ds(0, B)], sz0).wait()
        pltpu.make_async_copy(degbuf, deg_out.at[0, pl.ds(0, B)], sz1).wait()

    @pl.when(s == 0)
    def _():
        pltpu.sync_copy(gagg_sh, rs0.at[pl.ds(0, G)])
        pltpu.sync_copy(rs0.at[pl.ds(0, G)], gagg_out.at[c])


NB = 2000


def _d_body(agg_ref, deg_ref, x_ref, pos_ref, batch_ref, gf_ref,
            wn1_ref, wnx_ref, wnp_ref, wng_ref, bn_ref,
            gagg_ref, ecnt_ref, ncnt_ref,
            wg1_ref, wg2_ref, wg3_ref, bg_ref,
            unf_ref, nsum_ref, ugf_ref):
    x = x_ref[...]
    pos = pos_ref[...]
    aggsum = agg_ref[0] + agg_ref[1]
    deg = deg_ref[0, :, 0:1] + deg_ref[1, :, 0:1]
    agg_ef = aggsum / jnp.maximum(deg, 1.0)
    bt = batch_ref[0, 0, :]
    oh = (bt[:, None] == lax.broadcasted_iota(jnp.int32, (NB, G), 1)
          ).astype(jnp.float32)
    rep = jnp.dot(oh, gf_ref[...], preferred_element_type=jnp.float32)
    acc = (jnp.dot(agg_ef, wn1_ref[...], preferred_element_type=jnp.float32)
           + jnp.dot(x, wnx_ref[...], preferred_element_type=jnp.float32)
           + jnp.dot(pos, wnp_ref[...], preferred_element_type=jnp.float32)
           + jnp.dot(rep, wng_ref[...], preferred_element_type=jnp.float32)
           + bn_ref[...])
    unf = acc + jnp.concatenate([x, pos], axis=1)
    unf_ref[...] = unf

    @pl.when(pl.program_id(0) == 0)
    def _():
        nsum_ref[...] = jnp.zeros_like(nsum_ref)

    nsum_ref[...] += lax.dot_general(
        oh, unf, (((0,), (0,)), ((), ())),
        preferred_element_type=jnp.float32)

    @pl.when(pl.program_id(0) == pl.num_programs(0) - 1)
    def _():
        agg_nf = nsum_ref[...] / jnp.maximum(ncnt_ref[...], 1.0)
        gaggsum = gagg_ref[0] + gagg_ref[1]
        agg_efg = gaggsum / jnp.maximum(ecnt_ref[...], 1.0)
        gf = gf_ref[...]
        ugf_ref[...] = (
            jnp.dot(agg_nf, wg1_ref[...], preferred_element_type=jnp.float32)
            + jnp.dot(agg_efg, wg2_ref[...], preferred_element_type=jnp.float32)
            + jnp.dot(gf, wg3_ref[...], preferred_element_type=jnp.float32)
            + bg_ref[...] + gf)


def kernel(x, pos, edge_attr, global_feats, W_e, b_e, W_n, b_n, W_g, b_g,
           edge_index, batch):
    f32 = jnp.float32
    src = edge_index[0].astype(jnp.int32)
    dst = edge_index[1].astype(jnp.int32)
    batch = batch.astype(jnp.int32)

    nf_pad = jnp.concatenate(
        [x, pos, jnp.zeros((N, 5), f32)], axis=1)
    Wsp = jnp.concatenate([W_e[0:131], jnp.zeros((5, 128), f32)], axis=0)
    Wdp = jnp.concatenate([W_e[131:262], jnp.zeros((5, 128), f32)], axis=0)
    Gp = global_feats @ W_e[278:310] + b_e[None, :]
    W2 = jnp.concatenate([W_e[262:278], Gp], axis=0)
    starts = jnp.searchsorted(batch, jnp.arange(G, dtype=jnp.int32)
                              ).astype(jnp.int32)[None, :]
    ncnt = jnp.diff(jnp.concatenate(
        [starts[0], jnp.array([N], jnp.int32)])).astype(f32)[:, None]

    ps, pd = pl.pallas_call(
        _a1_body,
        out_shape=(jax.ShapeDtypeStruct((N, 128), f32),
                   jax.ShapeDtypeStruct((N, 128), f32)),
    )(nf_pad, Wsp, Wdp)

    src3 = src.reshape(E // EB, 1, EB)
    grid_a2 = (E // EB,)
    base, bsrc3, ecnt = pl.pallas_call(
        _a2_body,
        grid=grid_a2,
        in_specs=[
            pl.BlockSpec((EB, 16), lambda i: (i, 0)),
            pl.BlockSpec((1, 1, EB), lambda i: (i, 0, 0)),
            pl.BlockSpec((1, G), lambda i: (0, 0)),
            pl.BlockSpec((32, 128), lambda i: (0, 0)),
        ],
        out_specs=[
            pl.BlockSpec((EB, 128), lambda i: (i, 0)),
            pl.BlockSpec((1, 1, EB), lambda i: (i, 0, 0)),
            pl.BlockSpec((8, 128), lambda i: (0, 0)),
        ],
        out_shape=(jax.ShapeDtypeStruct((E, 128), f32),
                   jax.ShapeDtypeStruct((E // EB, 1, EB), jnp.int32),
                   jax.ShapeDtypeStruct((8, 128), f32)),
    )(edge_attr, src3, starts, W2)
    bsrc = bsrc3.reshape(E)
    ecnt_col = ecnt[0, 0:G][:, None]

    idx_pack = jnp.stack([src.reshape(E // B, B), dst.reshape(E // B, B),
                          bsrc.reshape(E // B, B)], axis=1)
    mesh = plsc.VectorSubcoreMesh(core_axis_name="c", subcore_axis_name="s",
                                  num_cores=NC, num_subcores=NS)
    sc_edge = functools.partial(
        pl.kernel, _sc_body,
        compiler_params=pltpu.CompilerParams(use_tc_tiling_on_sc=False),
        out_type=(jax.ShapeDtypeStruct((E, 128), f32),
                  jax.ShapeDtypeStruct((NC, NPAD, 128), f32),
                  jax.ShapeDtypeStruct((NC, NPAD, 16), f32),
                  jax.ShapeDtypeStruct((NC, G, 128), f32)),
        mesh=mesh,
        scratch_types=(
            [pltpu.VMEM((3, B), jnp.int32) for _ in range(4)]
            + [pltpu.VMEM((B, 128), f32) for _ in range(6)]
            + [pltpu.VMEM((B, 16), f32), pltpu.VMEM((B, 16), f32)]
            + [pltpu.VMEM_SHARED((NPAD, 128), f32),
               pltpu.VMEM_SHARED((NPAD, 16), f32),
               pltpu.VMEM_SHARED((G, 128), f32)]
            + [pltpu.SemaphoreType.DMA for _ in range(20)]
        ),
    )()
    updated_ef, agg2, deg2, gagg2 = sc_edge(ps, pd, base, idx_pack)

    batch3 = batch.reshape(N // NB, 1, NB)
    Wn1 = W_n[0:128]
    Wnx = W_n[128:256]
    Wnp = W_n[256:259]
    Wng = W_n[259:291]
    Wg1 = W_g[0:131]
    Wg2 = W_g[131:259]
    Wg3 = W_g[259:291]
    grid_d = (N // NB,)
    rep_spec = lambda shape: pl.BlockSpec(shape, lambda i: tuple(0 for _ in shape))
    updated_nf, _nsum, updated_gf = pl.pallas_call(
        _d_body,
        grid=grid_d,
        in_specs=[
            pl.BlockSpec((NC, NB, 128), lambda i: (0, i, 0)),
            pl.BlockSpec((NC, NB, 16), lambda i: (0, i, 0)),
            pl.BlockSpec((NB, 128), lambda i: (i, 0)),
            pl.BlockSpec((NB, 3), lambda i: (i, 0)),
            pl.BlockSpec((1, 1, NB), lambda i: (i, 0, 0)),
            rep_spec((G, 32)),
            rep_spec((128, 131)),
            rep_spec((128, 131)),
            rep_spec((3, 131)),
            rep_spec((32, 131)),
            rep_spec((1, 131)),
            rep_spec((NC, G, 128)),
            rep_spec((G, 1)),
            rep_spec((G, 1)),
            rep_spec((131, 32)),
            rep_spec((128, 32)),
            rep_spec((32, 32)),
            rep_spec((1, 32)),
        ],
        out_specs=[
            pl.BlockSpec((NB, 131), lambda i: (i, 0)),
            rep_spec((G, 131)),
            rep_spec((G, 32)),
        ],
        out_shape=(jax.ShapeDtypeStruct((N, 131), f32),
                   jax.ShapeDtypeStruct((G, 131), f32),
                   jax.ShapeDtypeStruct((G, 32), f32)),
    )(agg2, deg2, x, pos, batch3, global_feats,
      Wn1, Wnx, Wnp, Wng, b_n[None, :],
      gagg2, ecnt_col, ncnt,
      Wg1, Wg2, Wg3, b_g[None, :])

    return (updated_nf, updated_ef, updated_gf)
